# Initial kernel scaffold; baseline (speedup 1.0000x reference)
import jax
import jax.numpy as jnp
from jax.experimental import pallas as pl


def kernel(x, conv1_w, conv1_b, conv2_w, conv2_b, conv3_w, conv3_b, conv4_w, conv4_b, conv5_w, conv5_b, fc_w, fc_b):
    raise NotImplementedError("write your pallas kernel here")


# speedup vs baseline: 11.9967x; 11.9967x over previous
import functools

import jax
import jax.numpy as jnp
from jax.experimental import pallas as pl
from jax.experimental.pallas import tpu as pltpu


def _conv_taps(xf, w_ref, b, kh, kw, Ho, Wp, Wo):
    TB, _, Cin = xf.shape
    Cout = w_ref.shape[-1]
    L = Ho * Wp
    acc = jnp.zeros((TB * L, Cout), jnp.float32)
    for i in range(kh):
        for j in range(kw):
            off = i * Wp + j
            acc = acc + jnp.dot(
                xf[:, off:off + L, :].reshape(TB * L, Cin), w_ref[i * kw + j],
                preferred_element_type=jnp.float32)
    acc = jnp.maximum(acc + b, 0.0).astype(jnp.bfloat16)
    return acc.reshape(TB, Ho, Wp, Cout)[:, :, :Wo, :]


def _pool2x2(v):
    TB, H, W, C = v.shape
    v = v.reshape(TB, H // 2, 2, W // 2, 2 * C)
    h = jnp.maximum(v[:, :, 0], v[:, :, 1])
    return jnp.maximum(h[..., :C], h[..., C:])


def _pad_flat(v, p):
    TB, H, W, C = v.shape
    v = jnp.pad(v, ((0, 0), (p, p + 1), (p, p), (0, 0)))
    return v.reshape(TB, (H + 2 * p + 1) * (W + 2 * p), C)


def _stage_a_kernel(x_ref, w_ref, b_ref, o_ref):
    y = _conv_taps(x_ref[...], w_ref, b_ref[...], 3, 3, 8, 10, 8)
    o_ref[...] = _pool2x2(y)


def _stage_b_kernel(x_ref, w_ref, b_ref, o_ref):
    xf = _pad_flat(x_ref[...], 2)
    y = _conv_taps(xf, w_ref, b_ref[...], 5, 5, 4, 8, 4)
    h0 = jnp.maximum(y[:, 0], y[:, 1])
    h1 = jnp.maximum(y[:, 2], y[:, 3])
    o_ref[:, 0, 0, :] = jnp.maximum(h0[:, 0], h0[:, 1])
    o_ref[:, 0, 1, :] = jnp.maximum(h0[:, 2], h0[:, 3])
    o_ref[:, 1, 0, :] = jnp.maximum(h1[:, 0], h1[:, 1])
    o_ref[:, 1, 1, :] = jnp.maximum(h1[:, 2], h1[:, 3])


def _stage_c_kernel(x_ref, w3_ref, b3_ref, w4_ref, b4_ref, w5_ref, b5_ref,
                    wf_ref, bf_ref, o_ref):
    v = x_ref[...]
    for w_r, b_r in ((w3_ref, b3_ref), (w4_ref, b4_ref), (w5_ref, b5_ref)):
        xf = _pad_flat(v, 1)
        v = _conv_taps(xf, w_r, b_r[...], 3, 3, 2, 4, 2)
    p = jnp.maximum(jnp.maximum(v[:, 0, 0], v[:, 0, 1]),
                    jnp.maximum(v[:, 1, 0], v[:, 1, 1]))
    acc = jnp.dot(p, wf_ref[...], preferred_element_type=jnp.float32)
    o_ref[...] = acc + bf_ref[...]


def _batched_call(body, x, consts, out_tail, out_dtype, tb):
    B = x.shape[0]
    tb = min(tb, B)
    Bp = ((B + tb - 1) // tb) * tb
    if Bp != B:
        x = jnp.pad(x, ((0, Bp - B),) + ((0, 0),) * (x.ndim - 1))
    xdims = x.shape[1:]
    out = pl.pallas_call(
        body,
        out_shape=jax.ShapeDtypeStruct((Bp,) + out_tail, out_dtype),
        grid=(Bp // tb,),
        in_specs=[pl.BlockSpec((tb,) + xdims, lambda i: (i,) + (0,) * len(xdims))]
        + [pl.BlockSpec(c.shape, lambda i, _n=c.ndim: (0,) * _n) for c in consts],
        out_specs=pl.BlockSpec((tb,) + out_tail, lambda i: (i,) + (0,) * len(out_tail)),
        compiler_params=pltpu.CompilerParams(dimension_semantics=("parallel",)),
    )(x, *consts)
    return out[:B]


def kernel(x, conv1_w, conv1_b, conv2_w, conv2_b, conv3_w, conv3_b,
           conv4_w, conv4_b, conv5_w, conv5_b, fc_w, fc_b):
    B = x.shape[0]

    xs = x.reshape(B, 3, 8, 4, 8, 4).transpose(0, 2, 4, 3, 5, 1)
    xs = xs.reshape(B, 8, 8, 48).astype(jnp.bfloat16)
    xs = jnp.pad(xs, ((0, 0), (1, 2), (1, 1), (0, 0))).reshape(B, 110, 48)

    w1 = jnp.pad(conv1_w, ((2, 3), (2, 3), (0, 0), (0, 0)))
    w1 = w1.reshape(3, 4, 3, 4, 3, 128).transpose(0, 2, 1, 3, 4, 5)
    w1 = w1.reshape(9, 48, 128).astype(jnp.bfloat16)
    b1 = conv1_b.reshape(1, 128).astype(jnp.float32)

    w2 = conv2_w.reshape(25, 128, 192).astype(jnp.bfloat16)
    b2 = conv2_b.reshape(1, 192).astype(jnp.float32)
    w3 = conv3_w.reshape(9, 192, 384).astype(jnp.bfloat16)
    b3 = conv3_b.reshape(1, 384).astype(jnp.float32)
    w4 = conv4_w.reshape(9, 384, 256).astype(jnp.bfloat16)
    b4 = conv4_b.reshape(1, 256).astype(jnp.float32)
    w5 = conv5_w.reshape(9, 256, 256).astype(jnp.bfloat16)
    b5 = conv5_b.reshape(1, 256).astype(jnp.float32)
    NC = fc_w.shape[1]
    NCp = ((NC + 127) // 128) * 128
    wf = jnp.pad(fc_w, ((0, 0), (0, NCp - NC))).astype(jnp.bfloat16)
    bf = jnp.pad(fc_b, ((0, NCp - NC))).reshape(1, NCp).astype(jnp.float32)

    a = _batched_call(_stage_a_kernel, xs, (w1, b1), (4, 4, 128), jnp.bfloat16, 64)
    b_ = _batched_call(_stage_b_kernel, a, (w2, b2), (2, 2, 192), jnp.bfloat16, 128)
    out = _batched_call(_stage_c_kernel, b_, (w3, b3, w4, b4, w5, b5, wf, bf),
                        (NCp,), jnp.float32, 128)
    return out[:, :NC]


# speedup vs baseline: 28.1917x; 2.3500x over previous
import jax
import jax.numpy as jnp
from jax.experimental import pallas as pl
from jax.experimental.pallas import tpu as pltpu


def _sel(k, n_in, n_out, off):
    t = jnp.arange(k)[:, None, None]
    i = jnp.arange(n_in)[None, :, None]
    o = jnp.arange(n_out)[None, None, :]
    return (i == o + t - off).astype(jnp.float32)


def _dense_conv_w(w, hw_in, hw_out, c_in_pad, c_out_pad):
    kh, kw, ci, co = w.shape
    w = jnp.pad(w, ((0, 0), (0, 0), (0, c_in_pad - ci), (0, c_out_pad - co)))
    s = _sel(kh, hw_in, hw_out, kh // 2)
    full = jnp.einsum("ahx,bwy,abcd->hwcxyd", s, s, w)
    return full.reshape(hw_in * hw_in * c_in_pad, hw_out * hw_out * c_out_pad)


def _fused_kernel(x_ref, w1_ref, b1_ref, w2_ref, b2_ref, w3_ref, b3_ref,
                  w4_ref, b4_ref, w5_ref, b5_ref, wf_ref, bf_ref, o_ref):
    x = x_ref[...]

    def mm(v, w, b):
        acc = jnp.dot(v, w, preferred_element_type=jnp.float32) + b
        return jnp.maximum(acc, 0.0).astype(jnp.bfloat16)

    w1 = w1_ref[...]
    b1 = b1_ref[...]
    rows = [mm(x[:, ho * 512:(ho + 3) * 512], w1, b1) for ho in range(8)]

    p1 = []
    for h2 in range(4):
        h = jnp.maximum(rows[2 * h2], rows[2 * h2 + 1])
        p1 += [jnp.maximum(h[:, w * 256:w * 256 + 128],
                           h[:, w * 256 + 128:w * 256 + 256]) for w in range(4)]
    x2 = jnp.concatenate(p1, axis=1)

    b2 = b2_ref[...]
    r2 = [mm(x2, w2_ref[ho], b2) for ho in range(4)]

    p2 = []
    for h2 in range(2):
        h = jnp.maximum(r2[2 * h2], r2[2 * h2 + 1])
        p2 += [jnp.maximum(h[:, 0:256], h[:, 256:512]),
               jnp.maximum(h[:, 512:768], h[:, 768:1024])]
    x3 = jnp.concatenate(p2, axis=1)

    x4 = mm(x3, w3_ref[...], b3_ref[...])
    x5 = mm(x4, w4_ref[...], b4_ref[...])
    x6 = mm(x5, w5_ref[...], b5_ref[...])

    p3 = jnp.maximum(jnp.maximum(x6[:, 0:256], x6[:, 256:512]),
                     jnp.maximum(x6[:, 512:768], x6[:, 768:1024]))

    acc = jnp.dot(p3, wf_ref[...], preferred_element_type=jnp.float32)
    o_ref[...] = acc + bf_ref[...]


def kernel(x, conv1_w, conv1_b, conv2_w, conv2_b, conv3_w, conv3_b,
           conv4_w, conv4_b, conv5_w, conv5_b, fc_w, fc_b):
    B = x.shape[0]

    xs = x.reshape(B, 3, 8, 4, 8, 4).transpose(0, 2, 4, 3, 5, 1)
    xs = xs.reshape(B, 8, 8, 48).astype(jnp.bfloat16)
    xs = jnp.pad(xs, ((0, 0), (1, 1), (1, 1), (0, 0))).reshape(B, 10, 480)
    xs = jnp.pad(xs, ((0, 0), (0, 0), (0, 32))).reshape(B, 5120)

    w1p = jnp.pad(conv1_w, ((2, 3), (2, 3), (0, 0), (0, 0)))
    w1p = w1p.reshape(3, 4, 3, 4, 3, 128).transpose(0, 2, 1, 3, 4, 5)
    w1p = w1p.reshape(3, 3, 48, 128)
    sw1 = _sel(3, 10, 8, 0)
    w1r = jnp.einsum("bwy,abcd->awcyd", sw1, w1p).reshape(3, 480, 1024)
    w1r = jnp.pad(w1r, ((0, 0), (0, 32), (0, 0))).reshape(1536, 1024)
    w1r = w1r.astype(jnp.bfloat16)
    b1d = jnp.tile(conv1_b, 8).reshape(1, 1024).astype(jnp.float32)

    w2full = _dense_conv_w(conv2_w, 4, 4, 128, 256)
    w2s = w2full.reshape(2048, 4, 1024).transpose(1, 0, 2).astype(jnp.bfloat16)
    b2d = jnp.tile(jnp.pad(conv2_b, (0, 64)), 4).reshape(1, 1024)
    b2d = b2d.astype(jnp.float32)

    w3d = _dense_conv_w(conv3_w, 2, 2, 256, 384).astype(jnp.bfloat16)
    b3d = jnp.tile(conv3_b, 4).reshape(1, 1536).astype(jnp.float32)
    w4d = _dense_conv_w(conv4_w, 2, 2, 384, 256).astype(jnp.bfloat16)
    b4d = jnp.tile(conv4_b, 4).reshape(1, 1024).astype(jnp.float32)
    w5d = _dense_conv_w(conv5_w, 2, 2, 256, 256).astype(jnp.bfloat16)
    b5d = jnp.tile(conv5_b, 4).reshape(1, 1024).astype(jnp.float32)

    NC = fc_w.shape[1]
    NCp = ((NC + 127) // 128) * 128
    wf = jnp.pad(fc_w, ((0, 0), (0, NCp - NC))).astype(jnp.bfloat16)
    bf = jnp.pad(fc_b, ((0, NCp - NC))).reshape(1, NCp).astype(jnp.float32)

    consts = (w1r, b1d, w2s, b2d, w3d, b3d, w4d, b4d, w5d, b5d, wf, bf)
    TB = 256
    tb = min(TB, B)
    Bp = ((B + tb - 1) // tb) * tb
    if Bp != B:
        xs = jnp.pad(xs, ((0, Bp - B), (0, 0)))
    out = pl.pallas_call(
        _fused_kernel,
        out_shape=jax.ShapeDtypeStruct((Bp, NCp), jnp.float32),
        grid=(Bp // tb,),
        in_specs=[pl.BlockSpec((tb, 5120), lambda i: (i, 0))]
        + [pl.BlockSpec(c.shape, lambda i, _n=c.ndim: (0,) * _n) for c in consts],
        out_specs=pl.BlockSpec((tb, NCp), lambda i: (i, 0)),
        compiler_params=pltpu.CompilerParams(dimension_semantics=("parallel",)),
    )(xs, *consts)
    return out[:B, :NC]


# speedup vs baseline: 33.1908x; 1.1773x over previous
import jax
import jax.numpy as jnp
from jax.experimental import pallas as pl
from jax.experimental.pallas import tpu as pltpu


def _sel(k, n_in, n_out, stride, off):
    t = jnp.arange(k)[:, None, None]
    i = jnp.arange(n_in)[None, :, None]
    o = jnp.arange(n_out)[None, None, :]
    return (i == stride * o + t - off).astype(jnp.float32)


def _dense_conv_w(w, hw_in, hw_out, c_in_pad, c_out_pad, row_major_out=False):
    kh, kw, ci, co = w.shape
    w = jnp.pad(w, ((0, 0), (0, 0), (0, c_in_pad - ci), (0, c_out_pad - co)))
    s = _sel(kh, hw_in, hw_out, 1, kh // 2)
    if row_major_out:
        full = jnp.einsum("ahx,bwy,abcd->xhwcyd", s, s, w)
        return full.reshape(hw_out, hw_in * hw_in * c_in_pad,
                            hw_out * c_out_pad)
    full = jnp.einsum("ahx,bwy,abcd->hwcxyd", s, s, w)
    return full.reshape(hw_in * hw_in * c_in_pad, hw_out * hw_out * c_out_pad)


def _fused_kernel(x_ref, w1_ref, b1_ref, w2_ref, b2_ref, w3_ref, b3_ref,
                  w4_ref, b4_ref, w5_ref, b5_ref, wf_ref, bf_ref, o_ref):
    x = x_ref[...]

    def mm(v, w, b):
        acc = jnp.dot(v, w, preferred_element_type=jnp.float32) + b
        return jnp.maximum(acc, 0.0).astype(jnp.bfloat16)

    b1 = b1_ref[...]
    rows = []
    for ho in range(8):
        acc = jnp.dot(x[:, 0, 128 * ho:128 * ho + 256], w1_ref[0],
                      preferred_element_type=jnp.float32)
        acc += jnp.dot(x[:, 1, 128 * ho:128 * ho + 256], w1_ref[1],
                       preferred_element_type=jnp.float32)
        acc += jnp.dot(x[:, 2, 128 * ho:128 * ho + 256], w1_ref[2],
                       preferred_element_type=jnp.float32)
        rows.append(jnp.maximum(acc + b1, 0.0).astype(jnp.bfloat16))

    p1 = []
    for h2 in range(4):
        h = jnp.maximum(rows[2 * h2], rows[2 * h2 + 1])
        p1 += [jnp.maximum(h[:, w * 256:w * 256 + 128],
                           h[:, w * 256 + 128:w * 256 + 256]) for w in range(4)]
    x2 = jnp.concatenate(p1, axis=1)

    b2 = b2_ref[...]
    r2 = [mm(x2, w2_ref[ho], b2) for ho in range(4)]

    p2 = []
    for h2 in range(2):
        h = jnp.maximum(r2[2 * h2], r2[2 * h2 + 1])
        p2 += [jnp.maximum(h[:, 0:256], h[:, 256:512]),
               jnp.maximum(h[:, 512:768], h[:, 768:1024])]
    x3 = jnp.concatenate(p2, axis=1)

    x4 = mm(x3, w3_ref[...], b3_ref[...])
    x5 = mm(x4, w4_ref[...], b4_ref[...])
    x6 = mm(x5, w5_ref[...], b5_ref[...])

    p3 = jnp.maximum(jnp.maximum(x6[:, 0:256], x6[:, 256:512]),
                     jnp.maximum(x6[:, 512:768], x6[:, 768:1024]))

    acc = jnp.dot(p3, wf_ref[...], preferred_element_type=jnp.float32)
    o_ref[...] = acc + bf_ref[...]


def kernel(x, conv1_w, conv1_b, conv2_w, conv2_b, conv3_w, conv3_b,
           conv4_w, conv4_b, conv5_w, conv5_b, fc_w, fc_b):
    B = x.shape[0]

    xs = jnp.pad(x, ((0, 0), (0, 0), (2, 2), (0, 0)))
    xs = xs.reshape(B, 3, 36 * 32).astype(jnp.bfloat16)

    sw1 = _sel(7, 32, 8, 4, 2)
    w1d = jnp.einsum("bwy,abcd->cawyd", sw1, conv1_w)
    w1d = jnp.pad(w1d, ((0, 0), (0, 1), (0, 0), (0, 0), (0, 0)))
    w1d = w1d.reshape(3, 256, 1024).astype(jnp.bfloat16)
    b1d = jnp.tile(conv1_b, 8).reshape(1, 1024).astype(jnp.float32)

    w2s = _dense_conv_w(conv2_w, 4, 4, 128, 256, row_major_out=True)
    w2s = w2s.astype(jnp.bfloat16)
    b2d = jnp.tile(jnp.pad(conv2_b, (0, 64)), 4).reshape(1, 1024)
    b2d = b2d.astype(jnp.float32)

    w3d = _dense_conv_w(conv3_w, 2, 2, 256, 384).astype(jnp.bfloat16)
    b3d = jnp.tile(conv3_b, 4).reshape(1, 1536).astype(jnp.float32)
    w4d = _dense_conv_w(conv4_w, 2, 2, 384, 256).astype(jnp.bfloat16)
    b4d = jnp.tile(conv4_b, 4).reshape(1, 1024).astype(jnp.float32)
    w5d = _dense_conv_w(conv5_w, 2, 2, 256, 256).astype(jnp.bfloat16)
    b5d = jnp.tile(conv5_b, 4).reshape(1, 1024).astype(jnp.float32)

    NC = fc_w.shape[1]
    NCp = ((NC + 127) // 128) * 128
    wf = jnp.pad(fc_w, ((0, 0), (0, NCp - NC))).astype(jnp.bfloat16)
    bf = jnp.pad(fc_b, ((0, NCp - NC))).reshape(1, NCp).astype(jnp.float32)

    consts = (w1d, b1d, w2s, b2d, w3d, b3d, w4d, b4d, w5d, b5d, wf, bf)
    TB = 256
    tb = min(TB, B)
    Bp = ((B + tb - 1) // tb) * tb
    if Bp != B:
        xs = jnp.pad(xs, ((0, Bp - B), (0, 0), (0, 0)))
    out = pl.pallas_call(
        _fused_kernel,
        out_shape=jax.ShapeDtypeStruct((Bp, NCp), jnp.float32),
        grid=(Bp // tb,),
        in_specs=[pl.BlockSpec((tb, 3, 1152), lambda i: (i, 0, 0))]
        + [pl.BlockSpec(c.shape, lambda i, _n=c.ndim: (0,) * _n) for c in consts],
        out_specs=pl.BlockSpec((tb, NCp), lambda i: (i, 0)),
        compiler_params=pltpu.CompilerParams(dimension_semantics=("parallel",)),
    )(xs, *consts)
    return out[:B, :NC]


# speedup vs baseline: 36.9443x; 1.1131x over previous
import jax
import jax.numpy as jnp
from jax.experimental import pallas as pl
from jax.experimental.pallas import tpu as pltpu


def _dense_conv_w(w, hw_in, hw_out, c_in_pad, c_out_pad, row_major_out=False):
    kh, kw, ci, co = w.shape
    off, P = kh // 2, hw_out - 1
    wp = jnp.pad(w, ((P, P), (P, P), (0, c_in_pad - ci), (0, c_out_pad - co)))
    K = hw_in * hw_in * c_in_pad
    rows = []
    for ho in range(hw_out):
        hs = wp[off + P - ho:off + P - ho + hw_in]
        cols = [hs[:, off + P - wo:off + P - wo + hw_in] for wo in range(hw_out)]
        rows.append(jnp.stack(cols, axis=3))
    if row_major_out:
        return jnp.stack(rows, axis=0).reshape(hw_out, K, hw_out * c_out_pad)
    return jnp.stack(rows, axis=3).reshape(K, hw_out * hw_out * c_out_pad)


def _conv1_w(w):
    co = w.shape[-1]
    wt = w.transpose(2, 0, 1, 3)
    blocks = []
    for wo in range(8):
        lo = 4 * wo - 2
        s_lo, s_hi = max(0, -lo), min(7, 32 - lo)
        blk = jnp.pad(wt[:, :, s_lo:s_hi],
                      ((0, 0), (0, 1), (lo + s_lo, 32 - lo - s_hi), (0, 0)))
        blocks.append(blk)
    return jnp.stack(blocks, axis=3).reshape(3, 256, 8 * co)


def _fused_kernel(x_ref, w1_ref, b1_ref, w2_ref, b2_ref, w3_ref, b3_ref,
                  w4_ref, b4_ref, w5_ref, b5_ref, wf_ref, bf_ref, o_ref,
                  *, nc):
    xb = x_ref[...].astype(jnp.bfloat16)
    x = jnp.pad(xb, ((0, 0), (0, 0), (64, 64)))

    def mm(v, w, b):
        acc = jnp.dot(v, w, preferred_element_type=jnp.float32) + b
        return jnp.maximum(acc, 0.0).astype(jnp.bfloat16)

    b1 = b1_ref[...]
    rows = []
    for ho in range(8):
        acc = jnp.dot(x[:, 0, 128 * ho:128 * ho + 256], w1_ref[0],
                      preferred_element_type=jnp.float32)
        acc += jnp.dot(x[:, 1, 128 * ho:128 * ho + 256], w1_ref[1],
                       preferred_element_type=jnp.float32)
        acc += jnp.dot(x[:, 2, 128 * ho:128 * ho + 256], w1_ref[2],
                       preferred_element_type=jnp.float32)
        rows.append(jnp.maximum(acc + b1, 0.0).astype(jnp.bfloat16))

    p1 = []
    for h2 in range(4):
        h = jnp.maximum(rows[2 * h2], rows[2 * h2 + 1])
        p1 += [jnp.maximum(h[:, w * 256:w * 256 + 128],
                           h[:, w * 256 + 128:w * 256 + 256]) for w in range(4)]
    x2 = jnp.concatenate(p1, axis=1)

    b2 = b2_ref[...]
    r2 = [mm(x2, w2_ref[ho], b2) for ho in range(4)]

    p2 = []
    for h2 in range(2):
        h = jnp.maximum(r2[2 * h2], r2[2 * h2 + 1])
        p2 += [jnp.maximum(h[:, 0:256], h[:, 256:512]),
               jnp.maximum(h[:, 512:768], h[:, 768:1024])]
    x3 = jnp.concatenate(p2, axis=1)

    x4 = mm(x3, w3_ref[...], b3_ref[...])
    x5 = mm(x4, w4_ref[...], b4_ref[...])
    x6 = mm(x5, w5_ref[...], b5_ref[...])

    p3 = jnp.maximum(jnp.maximum(x6[:, 0:256], x6[:, 256:512]),
                     jnp.maximum(x6[:, 512:768], x6[:, 768:1024]))

    acc = jnp.dot(p3, wf_ref[...], preferred_element_type=jnp.float32)
    o_ref[...] = (acc + bf_ref[...])[:, :nc]


def kernel(x, conv1_w, conv1_b, conv2_w, conv2_b, conv3_w, conv3_b,
           conv4_w, conv4_b, conv5_w, conv5_b, fc_w, fc_b):
    import functools
    B = x.shape[0]
    xs = x.reshape(B, 3, 1024)

    w1d = _conv1_w(conv1_w).astype(jnp.bfloat16)
    b1d = jnp.tile(conv1_b, 8).reshape(1, 1024).astype(jnp.float32)

    w2s = _dense_conv_w(conv2_w, 4, 4, 128, 256, row_major_out=True)
    w2s = w2s.astype(jnp.bfloat16)
    b2d = jnp.tile(jnp.pad(conv2_b, (0, 64)), 4).reshape(1, 1024)
    b2d = b2d.astype(jnp.float32)

    w3d = _dense_conv_w(conv3_w, 2, 2, 256, 384).astype(jnp.bfloat16)
    b3d = jnp.tile(conv3_b, 4).reshape(1, 1536).astype(jnp.float32)
    w4d = _dense_conv_w(conv4_w, 2, 2, 384, 256).astype(jnp.bfloat16)
    b4d = jnp.tile(conv4_b, 4).reshape(1, 1024).astype(jnp.float32)
    w5d = _dense_conv_w(conv5_w, 2, 2, 256, 256).astype(jnp.bfloat16)
    b5d = jnp.tile(conv5_b, 4).reshape(1, 1024).astype(jnp.float32)

    NC = fc_w.shape[1]
    NCp = ((NC + 127) // 128) * 128
    wf = jnp.pad(fc_w, ((0, 0), (0, NCp - NC))).astype(jnp.bfloat16)
    bf = jnp.pad(fc_b, ((0, NCp - NC))).reshape(1, NCp).astype(jnp.float32)

    consts = (w1d, b1d, w2s, b2d, w3d, b3d, w4d, b4d, w5d, b5d, wf, bf)
    TB = 256
    tb = min(TB, B)
    Bp = ((B + tb - 1) // tb) * tb
    if Bp != B:
        xs = jnp.pad(xs, ((0, Bp - B), (0, 0), (0, 0)))
    out = pl.pallas_call(
        functools.partial(_fused_kernel, nc=NC),
        out_shape=jax.ShapeDtypeStruct((Bp, NC), jnp.float32),
        grid=(Bp // tb,),
        in_specs=[pl.BlockSpec((tb, 3, 1024), lambda i: (i, 0, 0))]
        + [pl.BlockSpec(c.shape, lambda i, _n=c.ndim: (0,) * _n) for c in consts],
        out_specs=pl.BlockSpec((tb, NC), lambda i: (i, 0)),
        compiler_params=pltpu.CompilerParams(dimension_semantics=("parallel",)),
    )(xs, *consts)
    return out[:B]
